# baseline (speedup 1.0000x reference)
import functools

import jax
import jax.numpy as jnp
from jax import lax
from jax.experimental import pallas as pl
from jax.experimental.pallas import tpu as pltpu
from jax.experimental.pallas import tpu_sc as plsc


def _sc_gather_slabs(artist_ids, genre_ids, atab4, gtab4):
    B = artist_ids.shape[0]
    info = plsc.get_sparse_core_info()
    NC, NS = info.num_cores, info.num_subcores
    NW = NC * NS
    bw = B // NW
    mesh = plsc.VectorSubcoreMesh(core_axis_name="c", subcore_axis_name="s")

    @functools.partial(
        pl.kernel,
        mesh=mesh,
        out_type=[
            jax.ShapeDtypeStruct((B, 128), jnp.float32),
            jax.ShapeDtypeStruct((B, 128), jnp.float32),
        ],
        scratch_types=[
            pltpu.VMEM((bw,), jnp.int32),
            pltpu.VMEM((bw,), jnp.int32),
            pltpu.VMEM((bw,), jnp.int32),
            pltpu.VMEM((bw,), jnp.int32),
            pltpu.VMEM((bw, 128), jnp.float32),
            pltpu.SemaphoreType.DMA,
        ],
    )
    def gather_k(aid_hbm, gid_hbm, atab_hbm, gtab_hbm, aout_hbm, gout_hbm,
                 aidx_v, gidx_v, aslab_v, gslab_v, slab_buf, sem):
        wid = lax.axis_index("s") * NC + lax.axis_index("c")
        base = wid * bw
        pltpu.sync_copy(aid_hbm.at[pl.ds(base, bw)], aidx_v)
        pltpu.sync_copy(gid_hbm.at[pl.ds(base, bw)], gidx_v)
        for k in range(bw // 16):
            sl = pl.ds(k * 16, 16)
            aslab_v[sl] = jnp.right_shift(aidx_v[sl], 2)
            gslab_v[sl] = jnp.right_shift(gidx_v[sl], 2)
        pltpu.async_copy(atab_hbm.at[aslab_v], slab_buf, sem).wait()
        pltpu.sync_copy(slab_buf, aout_hbm.at[pl.ds(base, bw)])
        pltpu.async_copy(gtab_hbm.at[gslab_v], slab_buf, sem).wait()
        pltpu.sync_copy(slab_buf, gout_hbm.at[pl.ds(base, bw)])

    return gather_k(artist_ids, genre_ids, atab4, gtab4)


def _tc_mlp(a4, g4, aid_col, gid_col, y_col, wy_row, by_row, w1a, w1g, w1y,
            b1_row, w2, b2_row):
    B = a4.shape[0]
    E = wy_row.shape[1]
    HID = w1a.shape[1]
    OUT = w2.shape[1]
    BLK = 2048

    def mlp_k(a4_ref, g4_ref, aid_ref, gid_ref, y_ref, wy_ref, by_ref,
              w1a_ref, w1g_ref, w1y_ref, b1_ref, w2_ref, b2_ref, o_ref):
        asub = jnp.bitwise_and(aid_ref[...], 3)
        gsub = jnp.bitwise_and(gid_ref[...], 3)
        a = jnp.zeros((BLK, E), jnp.float32)
        g = jnp.zeros((BLK, E), jnp.float32)
        for k in range(4):
            a = jnp.where(asub == k, a4_ref[:, k * E:(k + 1) * E], a)
            g = jnp.where(gsub == k, g4_ref[:, k * E:(k + 1) * E], g)
        y_emb = y_ref[...] * wy_ref[...] + by_ref[...]
        pre = (
            jnp.dot(a, w1a_ref[...], preferred_element_type=jnp.float32)
            + jnp.dot(g, w1g_ref[...], preferred_element_type=jnp.float32)
            + jnp.dot(y_emb, w1y_ref[...], preferred_element_type=jnp.float32)
            + b1_ref[...]
        )
        h = jnp.maximum(pre, 0.0)
        o_ref[...] = jnp.dot(h, w2_ref[...], preferred_element_type=jnp.float32) + b2_ref[...]

    return pl.pallas_call(
        mlp_k,
        grid=(B // BLK,),
        in_specs=[
            pl.BlockSpec((BLK, 128), lambda i: (i, 0)),
            pl.BlockSpec((BLK, 128), lambda i: (i, 0)),
            pl.BlockSpec((BLK, 1), lambda i: (i, 0)),
            pl.BlockSpec((BLK, 1), lambda i: (i, 0)),
            pl.BlockSpec((BLK, 1), lambda i: (i, 0)),
            pl.BlockSpec((1, E), lambda i: (0, 0)),
            pl.BlockSpec((1, E), lambda i: (0, 0)),
            pl.BlockSpec((E, HID), lambda i: (0, 0)),
            pl.BlockSpec((E, HID), lambda i: (0, 0)),
            pl.BlockSpec((E, HID), lambda i: (0, 0)),
            pl.BlockSpec((1, HID), lambda i: (0, 0)),
            pl.BlockSpec((HID, OUT), lambda i: (0, 0)),
            pl.BlockSpec((1, OUT), lambda i: (0, 0)),
        ],
        out_specs=pl.BlockSpec((BLK, OUT), lambda i: (i, 0)),
        out_shape=jax.ShapeDtypeStruct((B, OUT), jnp.float32),
    )(a4, g4, aid_col, gid_col, y_col, wy_row, by_row, w1a, w1g, w1y,
      b1_row, w2, b2_row)


def kernel(artist_ids, genre_ids, year_norms, artist_table, genre_table,
           Wy, by, W1, b1, W2, b2):
    E = artist_table.shape[1]
    aid = artist_ids.astype(jnp.int32)
    gid = genre_ids.astype(jnp.int32)
    atab4 = artist_table.reshape(artist_table.shape[0] // 4, 4 * E)
    gtab4 = genre_table.reshape(genre_table.shape[0] // 4, 4 * E)
    a4, g4 = _sc_gather_slabs(aid, gid, atab4, gtab4)
    y_col = year_norms[:, None]
    wy_row = Wy.T
    by_row = by[None, :]
    w1a = W1[:, :E].T
    w1g = W1[:, E:2 * E].T
    w1y = W1[:, 2 * E:3 * E].T
    b1_row = b1[None, :]
    w2 = W2.T
    b2_row = b2[None, :]
    return _tc_mlp(a4, g4, aid[:, None], gid[:, None], y_col, wy_row, by_row,
                   w1a, w1g, w1y, b1_row, w2, b2_row)


# speedup vs baseline: 1.6897x; 1.6897x over previous
import functools

import jax
import jax.numpy as jnp
from jax import lax
from jax.experimental import pallas as pl
from jax.experimental.pallas import tpu as pltpu
from jax.experimental.pallas import tpu_sc as plsc


def _tc_slabify(tabT):
    E, R = tabT.shape
    C = 8192
    nblk = (R + C - 1) // C

    def k(x_ref, o_ref):
        xt = jnp.swapaxes(x_ref[...], 0, 1)
        o_ref[...] = jnp.concatenate(
            [xt[q * (C // 4):(q + 1) * (C // 4), :] for q in range(4)], axis=1)

    return pl.pallas_call(
        k,
        grid=(nblk,),
        in_specs=[pl.BlockSpec((E, C), lambda i: (0, i))],
        out_specs=pl.BlockSpec((C // 4, 4 * E), lambda i: (i, 0)),
        out_shape=jax.ShapeDtypeStruct((nblk * (C // 4), 4 * E), jnp.float32),
    )(tabT)


def _sc_gather_slabs(artist_ids, genre_ids, atab4, gtab4):
    B = artist_ids.shape[0]
    info = plsc.get_sparse_core_info()
    NC, NS = info.num_cores, info.num_subcores
    NW = NC * NS
    bw = B // NW
    mesh = plsc.VectorSubcoreMesh(core_axis_name="c", subcore_axis_name="s")

    @functools.partial(
        pl.kernel,
        mesh=mesh,
        out_type=[
            jax.ShapeDtypeStruct((B, 128), jnp.float32),
            jax.ShapeDtypeStruct((B, 128), jnp.float32),
        ],
        scratch_types=[
            pltpu.VMEM((bw,), jnp.int32),
            pltpu.VMEM((bw,), jnp.int32),
            pltpu.VMEM((bw,), jnp.int32),
            pltpu.VMEM((bw,), jnp.int32),
            pltpu.VMEM((bw, 128), jnp.float32),
            pltpu.SemaphoreType.DMA,
        ],
    )
    def gather_k(aid_hbm, gid_hbm, atab_hbm, gtab_hbm, aout_hbm, gout_hbm,
                 aidx_v, gidx_v, aslab_v, gslab_v, slab_buf, sem):
        wid = lax.axis_index("s") * NC + lax.axis_index("c")
        base = wid * bw
        pltpu.sync_copy(aid_hbm.at[pl.ds(base, bw)], aidx_v)
        pltpu.sync_copy(gid_hbm.at[pl.ds(base, bw)], gidx_v)
        for k in range(bw // 16):
            sl = pl.ds(k * 16, 16)
            av = aidx_v[sl]
            aslab_v[sl] = (
                jnp.left_shift(jnp.right_shift(av, 13), 11)
                + jnp.bitwise_and(av, 2047)
            )
            gslab_v[sl] = jnp.right_shift(gidx_v[sl], 2)
        pltpu.async_copy(atab_hbm.at[aslab_v], slab_buf, sem).wait()
        pltpu.sync_copy(slab_buf, aout_hbm.at[pl.ds(base, bw)])
        pltpu.async_copy(gtab_hbm.at[gslab_v], slab_buf, sem).wait()
        pltpu.sync_copy(slab_buf, gout_hbm.at[pl.ds(base, bw)])

    return gather_k(artist_ids, genre_ids, atab4, gtab4)


def _tc_mlp(a4, g4, aid_col, gid_col, y_col, wy_row, by_row, w1a, w1g, w1y,
            b1_row, w2, b2_row):
    B = a4.shape[0]
    E = wy_row.shape[1]
    HID = w1a.shape[1]
    OUT = w2.shape[1]
    BLK = 2048

    def mlp_k(a4_ref, g4_ref, aid_ref, gid_ref, y_ref, wy_ref, by_ref,
              w1a_ref, w1g_ref, w1y_ref, b1_ref, w2_ref, b2_ref, o_ref):
        asub = jnp.bitwise_and(jnp.right_shift(aid_ref[...], 11), 3)
        gsub = jnp.bitwise_and(gid_ref[...], 3)
        a = jnp.zeros((BLK, E), jnp.float32)
        g = jnp.zeros((BLK, E), jnp.float32)
        for k in range(4):
            a = jnp.where(asub == k, a4_ref[:, k * E:(k + 1) * E], a)
            g = jnp.where(gsub == k, g4_ref[:, k * E:(k + 1) * E], g)
        y_emb = y_ref[...] * wy_ref[...] + by_ref[...]
        pre = (
            jnp.dot(a, w1a_ref[...], preferred_element_type=jnp.float32)
            + jnp.dot(g, w1g_ref[...], preferred_element_type=jnp.float32)
            + jnp.dot(y_emb, w1y_ref[...], preferred_element_type=jnp.float32)
            + b1_ref[...]
        )
        h = jnp.maximum(pre, 0.0)
        o_ref[...] = jnp.dot(h, w2_ref[...], preferred_element_type=jnp.float32) + b2_ref[...]

    return pl.pallas_call(
        mlp_k,
        grid=(B // BLK,),
        in_specs=[
            pl.BlockSpec((BLK, 128), lambda i: (i, 0)),
            pl.BlockSpec((BLK, 128), lambda i: (i, 0)),
            pl.BlockSpec((BLK, 1), lambda i: (i, 0)),
            pl.BlockSpec((BLK, 1), lambda i: (i, 0)),
            pl.BlockSpec((BLK, 1), lambda i: (i, 0)),
            pl.BlockSpec((1, E), lambda i: (0, 0)),
            pl.BlockSpec((1, E), lambda i: (0, 0)),
            pl.BlockSpec((E, HID), lambda i: (0, 0)),
            pl.BlockSpec((E, HID), lambda i: (0, 0)),
            pl.BlockSpec((E, HID), lambda i: (0, 0)),
            pl.BlockSpec((1, HID), lambda i: (0, 0)),
            pl.BlockSpec((HID, OUT), lambda i: (0, 0)),
            pl.BlockSpec((1, OUT), lambda i: (0, 0)),
        ],
        out_specs=pl.BlockSpec((BLK, OUT), lambda i: (i, 0)),
        out_shape=jax.ShapeDtypeStruct((B, OUT), jnp.float32),
    )(a4, g4, aid_col, gid_col, y_col, wy_row, by_row, w1a, w1g, w1y,
      b1_row, w2, b2_row)


def kernel(artist_ids, genre_ids, year_norms, artist_table, genre_table,
           Wy, by, W1, b1, W2, b2):
    E = artist_table.shape[1]
    aid = artist_ids.astype(jnp.int32)
    gid = genre_ids.astype(jnp.int32)
    atab4 = _tc_slabify(artist_table.T)
    gtab4 = genre_table.reshape(genre_table.shape[0] // 4, 4 * E)
    a4, g4 = _sc_gather_slabs(aid, gid, atab4, gtab4)
    y_col = year_norms[:, None]
    wy_row = Wy.T
    by_row = by[None, :]
    w1a = W1[:, :E].T
    w1g = W1[:, E:2 * E].T
    w1y = W1[:, 2 * E:3 * E].T
    b1_row = b1[None, :]
    w2 = W2.T
    b2_row = b2[None, :]
    return _tc_mlp(a4, g4, aid[:, None], gid[:, None], y_col, wy_row, by_row,
                   w1a, w1g, w1y, b1_row, w2, b2_row)


# speedup vs baseline: 2.2581x; 1.3364x over previous
import functools

import jax
import jax.numpy as jnp
from jax import lax
from jax.experimental import pallas as pl
from jax.experimental.pallas import tpu as pltpu
from jax.experimental.pallas import tpu_sc as plsc


def _tc_slabify_bf16(tabT):
    E, R = tabT.shape
    C = 8192
    nblk = (R + C - 1) // C

    def k(x_ref, o_ref):
        xb = x_ref[...].astype(jnp.bfloat16)
        xt = jnp.swapaxes(xb, 0, 1)
        xp = pltpu.bitcast(xt, jnp.float32)
        o_ref[...] = jnp.concatenate(
            [xp[q * (C // 8):(q + 1) * (C // 8), :] for q in range(4)], axis=1)

    return pl.pallas_call(
        k,
        grid=(nblk,),
        in_specs=[pl.BlockSpec((E, C), lambda i: (0, i))],
        out_specs=pl.BlockSpec((C // 8, 4 * E), lambda i: (i, 0)),
        out_shape=jax.ShapeDtypeStruct((nblk * (C // 8), 4 * E), jnp.float32),
    )(tabT)


def _sc_gather_slabs(artist_ids, genre_ids, atab4, gtab4):
    B = artist_ids.shape[0]
    info = plsc.get_sparse_core_info()
    NC, NS = info.num_cores, info.num_subcores
    NW = NC * NS
    bw = B // NW
    mesh = plsc.VectorSubcoreMesh(core_axis_name="c", subcore_axis_name="s")

    @functools.partial(
        pl.kernel,
        mesh=mesh,
        out_type=[
            jax.ShapeDtypeStruct((B, 128), jnp.float32),
            jax.ShapeDtypeStruct((B, 128), jnp.float32),
        ],
        scratch_types=[
            pltpu.VMEM((bw,), jnp.int32),
            pltpu.VMEM((bw,), jnp.int32),
            pltpu.VMEM((bw,), jnp.int32),
            pltpu.VMEM((bw,), jnp.int32),
            pltpu.VMEM((bw, 128), jnp.float32),
            pltpu.SemaphoreType.DMA,
        ],
    )
    def gather_k(aid_hbm, gid_hbm, atab_hbm, gtab_hbm, aout_hbm, gout_hbm,
                 aidx_v, gidx_v, aslab_v, gslab_v, slab_buf, sem):
        wid = lax.axis_index("s") * NC + lax.axis_index("c")
        base = wid * bw
        pltpu.sync_copy(aid_hbm.at[pl.ds(base, bw)], aidx_v)
        pltpu.sync_copy(gid_hbm.at[pl.ds(base, bw)], gidx_v)
        for k in range(bw // 16):
            sl = pl.ds(k * 16, 16)
            av = aidx_v[sl]
            aslab_v[sl] = (
                jnp.left_shift(jnp.right_shift(av, 13), 10)
                + jnp.bitwise_and(jnp.right_shift(av, 1), 1023)
            )
            gslab_v[sl] = jnp.right_shift(gidx_v[sl], 2)
        pltpu.async_copy(atab_hbm.at[aslab_v], slab_buf, sem).wait()
        pltpu.sync_copy(slab_buf, aout_hbm.at[pl.ds(base, bw)])
        pltpu.async_copy(gtab_hbm.at[gslab_v], slab_buf, sem).wait()
        pltpu.sync_copy(slab_buf, gout_hbm.at[pl.ds(base, bw)])

    return gather_k(artist_ids, genre_ids, atab4, gtab4)


def _tc_mlp(a4, g4, aid_col, gid_col, y_col, wy_row, by_row, w1a, w1g, w1y,
            b1_row, w2, b2_row):
    B = a4.shape[0]
    E = wy_row.shape[1]
    HID = w1a.shape[1]
    OUT = w2.shape[1]
    BLK = 2048

    def mlp_k(a4_ref, g4_ref, aid_ref, gid_ref, y_ref, wy_ref, by_ref,
              w1a_ref, w1g_ref, w1y_ref, b1_ref, w2_ref, b2_ref, o_ref):
        aid = aid_ref[...]
        asub = jnp.bitwise_and(jnp.right_shift(aid, 11), 3)
        parity = jnp.bitwise_and(aid, 1)
        gsub = jnp.bitwise_and(gid_ref[...], 3)
        ap = jnp.zeros((BLK, E), jnp.float32)
        g = jnp.zeros((BLK, E), jnp.float32)
        for k in range(4):
            ap = jnp.where(asub == k, a4_ref[:, k * E:(k + 1) * E], ap)
            g = jnp.where(gsub == k, g4_ref[:, k * E:(k + 1) * E], g)
        bits = pltpu.bitcast(ap, jnp.uint32)
        lo = pltpu.bitcast(jnp.left_shift(bits, 16), jnp.float32)
        hi = pltpu.bitcast(
            jnp.bitwise_and(bits, jnp.uint32(0xFFFF0000)), jnp.float32)
        a = jnp.where(parity == 1, hi, lo)
        y_emb = y_ref[...] * wy_ref[...] + by_ref[...]
        pre = (
            jnp.dot(a, w1a_ref[...], preferred_element_type=jnp.float32)
            + jnp.dot(g, w1g_ref[...], preferred_element_type=jnp.float32)
            + jnp.dot(y_emb, w1y_ref[...], preferred_element_type=jnp.float32)
            + b1_ref[...]
        )
        h = jnp.maximum(pre, 0.0)
        o_ref[...] = jnp.dot(h, w2_ref[...], preferred_element_type=jnp.float32) + b2_ref[...]

    return pl.pallas_call(
        mlp_k,
        grid=(B // BLK,),
        in_specs=[
            pl.BlockSpec((BLK, 128), lambda i: (i, 0)),
            pl.BlockSpec((BLK, 128), lambda i: (i, 0)),
            pl.BlockSpec((BLK, 1), lambda i: (i, 0)),
            pl.BlockSpec((BLK, 1), lambda i: (i, 0)),
            pl.BlockSpec((BLK, 1), lambda i: (i, 0)),
            pl.BlockSpec((1, E), lambda i: (0, 0)),
            pl.BlockSpec((1, E), lambda i: (0, 0)),
            pl.BlockSpec((E, HID), lambda i: (0, 0)),
            pl.BlockSpec((E, HID), lambda i: (0, 0)),
            pl.BlockSpec((E, HID), lambda i: (0, 0)),
            pl.BlockSpec((1, HID), lambda i: (0, 0)),
            pl.BlockSpec((HID, OUT), lambda i: (0, 0)),
            pl.BlockSpec((1, OUT), lambda i: (0, 0)),
        ],
        out_specs=pl.BlockSpec((BLK, OUT), lambda i: (i, 0)),
        out_shape=jax.ShapeDtypeStruct((B, OUT), jnp.float32),
    )(a4, g4, aid_col, gid_col, y_col, wy_row, by_row, w1a, w1g, w1y,
      b1_row, w2, b2_row)


def kernel(artist_ids, genre_ids, year_norms, artist_table, genre_table,
           Wy, by, W1, b1, W2, b2):
    E = artist_table.shape[1]
    aid = artist_ids.astype(jnp.int32)
    gid = genre_ids.astype(jnp.int32)
    atab4 = _tc_slabify_bf16(artist_table.T)
    gtab4 = genre_table.reshape(genre_table.shape[0] // 4, 4 * E)
    a4, g4 = _sc_gather_slabs(aid, gid, atab4, gtab4)
    y_col = year_norms[:, None]
    wy_row = Wy.T
    by_row = by[None, :]
    w1a = W1[:, :E].T
    w1g = W1[:, E:2 * E].T
    w1y = W1[:, 2 * E:3 * E].T
    b1_row = b1[None, :]
    w2 = W2.T
    b2_row = b2[None, :]
    return _tc_mlp(a4, g4, aid[:, None], gid[:, None], y_col, wy_row, by_row,
                   w1a, w1g, w1y, b1_row, w2, b2_row)


# speedup vs baseline: 2.5833x; 1.1440x over previous
import functools

import jax
import jax.numpy as jnp
from jax import lax
from jax.experimental import pallas as pl
from jax.experimental.pallas import tpu as pltpu
from jax.experimental.pallas import tpu_sc as plsc


def _tc_slabify_bf16(tabT):
    E, R = tabT.shape
    C = 16384
    nblk = (R + C - 1) // C

    def k(x_ref, o_ref):
        xb = x_ref[...].astype(jnp.bfloat16)
        xt = jnp.swapaxes(xb, 0, 1)
        xp = pltpu.bitcast(xt, jnp.float32)
        o_ref[...] = jnp.concatenate(
            [xp[q * (C // 8):(q + 1) * (C // 8), :] for q in range(4)], axis=1)

    return pl.pallas_call(
        k,
        grid=(nblk,),
        in_specs=[pl.BlockSpec((E, C), lambda i: (0, i))],
        out_specs=pl.BlockSpec((C // 8, 4 * E), lambda i: (i, 0)),
        out_shape=jax.ShapeDtypeStruct((nblk * (C // 8), 4 * E), jnp.float32),
    )(tabT)


def _sc_gather_slabs(artist_ids, genre_ids, atab4, gtab4):
    B = artist_ids.shape[0]
    info = plsc.get_sparse_core_info()
    NC, NS = info.num_cores, info.num_subcores
    NW = NC * NS
    bw = B // NW
    hw = bw // 2
    mesh = plsc.VectorSubcoreMesh(core_axis_name="c", subcore_axis_name="s")

    @functools.partial(
        pl.kernel,
        mesh=mesh,
        out_type=[
            jax.ShapeDtypeStruct((B, 128), jnp.float32),
            jax.ShapeDtypeStruct((B, 128), jnp.float32),
        ],
        scratch_types=[
            pltpu.VMEM((bw,), jnp.int32),
            pltpu.VMEM((bw,), jnp.int32),
            pltpu.VMEM((bw,), jnp.int32),
            pltpu.VMEM((bw,), jnp.int32),
            pltpu.VMEM((bw, 128), jnp.float32),
            pltpu.VMEM((hw, 128), jnp.float32),
            pltpu.SemaphoreType.DMA,
            pltpu.SemaphoreType.DMA,
        ],
    )
    def gather_k(aid_hbm, gid_hbm, atab_hbm, gtab_hbm, aout_hbm, gout_hbm,
                 aidx_v, gidx_v, aslab_v, gslab_v, abuf, gbuf, sem_a, sem_g):
        wid = lax.axis_index("s") * NC + lax.axis_index("c")
        base = wid * bw
        pltpu.sync_copy(aid_hbm.at[pl.ds(base, bw)], aidx_v)
        pltpu.sync_copy(gid_hbm.at[pl.ds(base, bw)], gidx_v)
        for k in range(bw // 16):
            sl = pl.ds(k * 16, 16)
            av = aidx_v[sl]
            aslab_v[sl] = (
                jnp.left_shift(jnp.right_shift(av, 14), 11)
                + jnp.bitwise_and(jnp.right_shift(av, 1), 2047)
            )
            gslab_v[sl] = jnp.right_shift(gidx_v[sl], 2)
        ca = pltpu.async_copy(atab_hbm.at[aslab_v], abuf, sem_a)
        for h in range(2):
            pltpu.async_copy(
                gtab_hbm.at[gslab_v.at[pl.ds(h * hw, hw)]], gbuf, sem_g
            ).wait()
            pltpu.sync_copy(gbuf, gout_hbm.at[pl.ds(base + h * hw, hw)])
        ca.wait()
        pltpu.sync_copy(abuf, aout_hbm.at[pl.ds(base, bw)])

    return gather_k(artist_ids, genre_ids, atab4, gtab4)


def _tc_mlp(a4, g4, meta, y2, wy_row, by_row, w1a, w1g, w1y,
            b1_row, w2, b2_row):
    B = a4.shape[0]
    E = wy_row.shape[1]
    HID = w1a.shape[1]
    OUT = w2.shape[1]
    BLK = 2048

    def mlp_k(a4_ref, g4_ref, meta_ref, y_ref, wy_ref, by_ref,
              w1a_ref, w1g_ref, w1y_ref, b1_ref, w2_ref, b2_ref, o_ref):
        m = jnp.swapaxes(meta_ref[...].reshape(1, BLK), 0, 1)
        asub = jnp.bitwise_and(m, 3)
        parity = jnp.bitwise_and(jnp.right_shift(m, 2), 1)
        gsub = jnp.bitwise_and(jnp.right_shift(m, 3), 3)
        ap = jnp.zeros((BLK, E), jnp.float32)
        g = jnp.zeros((BLK, E), jnp.float32)
        for k in range(4):
            ap = jnp.where(asub == k, a4_ref[:, k * E:(k + 1) * E], ap)
            g = jnp.where(gsub == k, g4_ref[:, k * E:(k + 1) * E], g)
        bits = pltpu.bitcast(ap, jnp.uint32)
        lo = pltpu.bitcast(jnp.left_shift(bits, 16), jnp.float32)
        hi = pltpu.bitcast(
            jnp.bitwise_and(bits, jnp.uint32(0xFFFF0000)), jnp.float32)
        a = jnp.where(parity == 1, hi, lo)
        y_col = jnp.swapaxes(y_ref[...].reshape(1, BLK), 0, 1)
        y_emb = y_col * wy_ref[...] + by_ref[...]
        pre = (
            jnp.dot(a, w1a_ref[...], preferred_element_type=jnp.float32)
            + jnp.dot(g, w1g_ref[...], preferred_element_type=jnp.float32)
            + jnp.dot(y_emb, w1y_ref[...], preferred_element_type=jnp.float32)
            + b1_ref[...]
        )
        h = jnp.maximum(pre, 0.0)
        o_ref[...] = jnp.dot(h, w2_ref[...], preferred_element_type=jnp.float32) + b2_ref[...]

    return pl.pallas_call(
        mlp_k,
        grid=(B // BLK,),
        in_specs=[
            pl.BlockSpec((BLK, 128), lambda i: (i, 0)),
            pl.BlockSpec((BLK, 128), lambda i: (i, 0)),
            pl.BlockSpec((1, 1, BLK), lambda i: (i, 0, 0)),
            pl.BlockSpec((1, 1, BLK), lambda i: (i, 0, 0)),
            pl.BlockSpec((1, E), lambda i: (0, 0)),
            pl.BlockSpec((1, E), lambda i: (0, 0)),
            pl.BlockSpec((E, HID), lambda i: (0, 0)),
            pl.BlockSpec((E, HID), lambda i: (0, 0)),
            pl.BlockSpec((E, HID), lambda i: (0, 0)),
            pl.BlockSpec((1, HID), lambda i: (0, 0)),
            pl.BlockSpec((HID, OUT), lambda i: (0, 0)),
            pl.BlockSpec((1, OUT), lambda i: (0, 0)),
        ],
        out_specs=pl.BlockSpec((BLK, OUT), lambda i: (i, 0)),
        out_shape=jax.ShapeDtypeStruct((B, OUT), jnp.float32),
    )(a4, g4, meta, y2, wy_row, by_row, w1a, w1g, w1y, b1_row, w2, b2_row)


def kernel(artist_ids, genre_ids, year_norms, artist_table, genre_table,
           Wy, by, W1, b1, W2, b2):
    E = artist_table.shape[1]
    B = artist_ids.shape[0]
    BLK = 2048
    aid = artist_ids.astype(jnp.int32)
    gid = genre_ids.astype(jnp.int32)
    atab4 = _tc_slabify_bf16(artist_table.T)
    gtab4 = genre_table.reshape(genre_table.shape[0] // 4, 4 * E)
    a4, g4 = _sc_gather_slabs(aid, gid, atab4, gtab4)
    meta = (
        jnp.bitwise_and(jnp.right_shift(aid, 12), 3)
        | jnp.left_shift(jnp.bitwise_and(aid, 1), 2)
        | jnp.left_shift(jnp.bitwise_and(gid, 3), 3)
    ).reshape(B // BLK, 1, BLK)
    y2 = year_norms.reshape(B // BLK, 1, BLK)
    wy_row = Wy.T
    by_row = by[None, :]
    w1a = W1[:, :E].T
    w1g = W1[:, E:2 * E].T
    w1y = W1[:, 2 * E:3 * E].T
    b1_row = b1[None, :]
    w2 = W2.T
    b2_row = b2[None, :]
    return _tc_mlp(a4, g4, meta, y2, wy_row, by_row, w1a, w1g, w1y,
                   b1_row, w2, b2_row)


# speedup vs baseline: 2.6462x; 1.0243x over previous
import functools

import jax
import jax.numpy as jnp
from jax import lax
from jax.experimental import pallas as pl
from jax.experimental.pallas import tpu as pltpu
from jax.experimental.pallas import tpu_sc as plsc


def _tc_slabify_bf16(tabT):
    E, R = tabT.shape
    C = 16384
    nblk = (R + C - 1) // C

    def k(x_ref, o_ref):
        xb = x_ref[...].astype(jnp.bfloat16)
        xt = jnp.swapaxes(xb, 0, 1)
        xp = pltpu.bitcast(xt, jnp.float32)
        o_ref[...] = jnp.concatenate(
            [xp[q * (C // 8):(q + 1) * (C // 8), :] for q in range(4)], axis=1)

    return pl.pallas_call(
        k,
        grid=(nblk,),
        in_specs=[pl.BlockSpec((E, C), lambda i: (0, i))],
        out_specs=pl.BlockSpec((C // 8, 4 * E), lambda i: (i, 0)),
        out_shape=jax.ShapeDtypeStruct((nblk * (C // 8), 4 * E), jnp.float32),
    )(tabT)


def _sc_gather_slabs(artist_ids, genre_ids, atab4, gtab4):
    B = artist_ids.shape[0]
    info = plsc.get_sparse_core_info()
    NC, NS = info.num_cores, info.num_subcores
    NW = NC * NS
    bw = B // NW
    hw = bw // 2
    mesh = plsc.VectorSubcoreMesh(core_axis_name="c", subcore_axis_name="s")

    @functools.partial(
        pl.kernel,
        mesh=mesh,
        out_type=[
            jax.ShapeDtypeStruct((B, 128), jnp.float32),
            jax.ShapeDtypeStruct((B, 128), jnp.float32),
        ],
        scratch_types=[
            pltpu.VMEM((bw,), jnp.int32),
            pltpu.VMEM((bw,), jnp.int32),
            pltpu.VMEM((bw,), jnp.int32),
            pltpu.VMEM((bw,), jnp.int32),
            pltpu.VMEM((bw, 128), jnp.float32),
            pltpu.VMEM((hw, 128), jnp.float32),
            pltpu.SemaphoreType.DMA,
            pltpu.SemaphoreType.DMA,
        ],
    )
    def gather_k(aid_hbm, gid_hbm, atab_hbm, gtab_hbm, aout_hbm, gout_hbm,
                 aidx_v, gidx_v, aslab_v, gslab_v, abuf, gbuf, sem_a, sem_g):
        wid = lax.axis_index("s") * NC + lax.axis_index("c")
        base = wid * bw
        pltpu.sync_copy(aid_hbm.at[pl.ds(base, bw)], aidx_v)
        pltpu.sync_copy(gid_hbm.at[pl.ds(base, bw)], gidx_v)
        for k in range(bw // 16):
            sl = pl.ds(k * 16, 16)
            av = aidx_v[sl]
            aslab_v[sl] = (
                jnp.left_shift(jnp.right_shift(av, 14), 11)
                + jnp.bitwise_and(jnp.right_shift(av, 1), 2047)
            )
            gslab_v[sl] = jnp.right_shift(gidx_v[sl], 2)
        ca = pltpu.async_copy(atab_hbm.at[aslab_v], abuf, sem_a)
        for h in range(2):
            pltpu.async_copy(
                gtab_hbm.at[gslab_v.at[pl.ds(h * hw, hw)]], gbuf, sem_g
            ).wait()
            pltpu.sync_copy(gbuf, gout_hbm.at[pl.ds(base + h * hw, hw)])
        ca.wait()
        pltpu.sync_copy(abuf, aout_hbm.at[pl.ds(base, bw)])

    return gather_k(artist_ids, genre_ids, atab4, gtab4)


def _tc_mlp(a4, g4, meta, y2, wy_row, by_row, w1a, w1g, w1y,
            b1_row, w2, b2_row):
    B = a4.shape[0]
    E = wy_row.shape[1]
    HID = w1a.shape[1]
    OUT = w2.shape[1]
    BLK = 2048

    def mlp_k(a4_ref, g4_ref, meta_ref, y_ref, wy_ref, by_ref,
              w1a_ref, w1g_ref, w1y_ref, b1_ref, w2_ref, b2_ref, o_ref):
        m = jnp.swapaxes(meta_ref[...].reshape(1, BLK), 0, 1)
        asub = jnp.bitwise_and(m, 3)
        parity = jnp.bitwise_and(jnp.right_shift(m, 2), 1)
        gsub = jnp.bitwise_and(jnp.right_shift(m, 3), 3)
        ap = jnp.zeros((BLK, E), jnp.float32)
        g = jnp.zeros((BLK, E), jnp.float32)
        for k in range(4):
            ap = jnp.where(asub == k, a4_ref[:, k * E:(k + 1) * E], ap)
            g = jnp.where(gsub == k, g4_ref[:, k * E:(k + 1) * E], g)
        bits = pltpu.bitcast(ap, jnp.uint32)
        lo = pltpu.bitcast(jnp.left_shift(bits, 16), jnp.float32)
        hi = pltpu.bitcast(
            jnp.bitwise_and(bits, jnp.uint32(0xFFFF0000)), jnp.float32)
        a = jnp.where(parity == 1, hi, lo).astype(jnp.bfloat16)
        gb = g.astype(jnp.bfloat16)
        y_col = jnp.swapaxes(y_ref[...].reshape(1, BLK), 0, 1)
        y_emb = (y_col * wy_ref[...] + by_ref[...]).astype(jnp.bfloat16)
        pre = (
            jnp.dot(a, w1a_ref[...], preferred_element_type=jnp.float32)
            + jnp.dot(gb, w1g_ref[...], preferred_element_type=jnp.float32)
            + jnp.dot(y_emb, w1y_ref[...], preferred_element_type=jnp.float32)
            + b1_ref[...]
        )
        h = jnp.maximum(pre, 0.0).astype(jnp.bfloat16)
        o_ref[...] = jnp.dot(h, w2_ref[...], preferred_element_type=jnp.float32) + b2_ref[...]

    return pl.pallas_call(
        mlp_k,
        grid=(B // BLK,),
        in_specs=[
            pl.BlockSpec((BLK, 128), lambda i: (i, 0)),
            pl.BlockSpec((BLK, 128), lambda i: (i, 0)),
            pl.BlockSpec((1, 1, BLK), lambda i: (i, 0, 0)),
            pl.BlockSpec((1, 1, BLK), lambda i: (i, 0, 0)),
            pl.BlockSpec((1, E), lambda i: (0, 0)),
            pl.BlockSpec((1, E), lambda i: (0, 0)),
            pl.BlockSpec((E, HID), lambda i: (0, 0)),
            pl.BlockSpec((E, HID), lambda i: (0, 0)),
            pl.BlockSpec((E, HID), lambda i: (0, 0)),
            pl.BlockSpec((1, HID), lambda i: (0, 0)),
            pl.BlockSpec((HID, OUT), lambda i: (0, 0)),
            pl.BlockSpec((1, OUT), lambda i: (0, 0)),
        ],
        out_specs=pl.BlockSpec((BLK, OUT), lambda i: (i, 0)),
        out_shape=jax.ShapeDtypeStruct((B, OUT), jnp.float32),
    )(a4, g4, meta, y2, wy_row, by_row, w1a, w1g, w1y, b1_row, w2, b2_row)


def kernel(artist_ids, genre_ids, year_norms, artist_table, genre_table,
           Wy, by, W1, b1, W2, b2):
    E = artist_table.shape[1]
    B = artist_ids.shape[0]
    BLK = 2048
    aid = artist_ids.astype(jnp.int32)
    gid = genre_ids.astype(jnp.int32)
    atab4 = _tc_slabify_bf16(artist_table.T)
    gtab4 = genre_table.reshape(genre_table.shape[0] // 4, 4 * E)
    a4, g4 = _sc_gather_slabs(aid, gid, atab4, gtab4)
    meta = (
        jnp.bitwise_and(jnp.right_shift(aid, 12), 3)
        | jnp.left_shift(jnp.bitwise_and(aid, 1), 2)
        | jnp.left_shift(jnp.bitwise_and(gid, 3), 3)
    ).reshape(B // BLK, 1, BLK)
    y2 = year_norms.reshape(B // BLK, 1, BLK)
    wy_row = Wy.T
    by_row = by[None, :]
    w1a = W1[:, :E].T.astype(jnp.bfloat16)
    w1g = W1[:, E:2 * E].T.astype(jnp.bfloat16)
    w1y = W1[:, 2 * E:3 * E].T.astype(jnp.bfloat16)
    b1_row = b1[None, :]
    w2 = W2.T.astype(jnp.bfloat16)
    b2_row = b2[None, :]
    return _tc_mlp(a4, g4, meta, y2, wy_row, by_row, w1a, w1g, w1y,
                   b1_row, w2, b2_row)


# speedup vs baseline: 2.7110x; 1.0245x over previous
import functools

import jax
import jax.numpy as jnp
from jax import lax
from jax.experimental import pallas as pl
from jax.experimental.pallas import tpu as pltpu
from jax.experimental.pallas import tpu_sc as plsc


def _tc_slabify_bf16(tabT):
    E, R = tabT.shape
    C = 32768
    nblk = (R + C - 1) // C

    def k(x_ref, o_ref):
        xb = x_ref[...].astype(jnp.bfloat16)
        xt = jnp.swapaxes(xb, 0, 1)
        xp = pltpu.bitcast(xt, jnp.float32)
        o_ref[...] = jnp.concatenate(
            [xp[q * (C // 8):(q + 1) * (C // 8), :] for q in range(4)], axis=1)

    return pl.pallas_call(
        k,
        grid=(nblk,),
        in_specs=[pl.BlockSpec((E, C), lambda i: (0, i))],
        out_specs=pl.BlockSpec((C // 8, 4 * E), lambda i: (i, 0)),
        out_shape=jax.ShapeDtypeStruct((nblk * (C // 8), 4 * E), jnp.float32),
    )(tabT)


def _sc_gather_slabs(artist_ids, genre_ids, atab4, gtab4):
    B = artist_ids.shape[0]
    info = plsc.get_sparse_core_info()
    NC, NS = info.num_cores, info.num_subcores
    NW = NC * NS
    bw = B // NW
    hw = bw // 2
    mesh = plsc.VectorSubcoreMesh(core_axis_name="c", subcore_axis_name="s")

    @functools.partial(
        pl.kernel,
        mesh=mesh,
        out_type=[
            jax.ShapeDtypeStruct((B, 128), jnp.float32),
            jax.ShapeDtypeStruct((B, 128), jnp.float32),
        ],
        scratch_types=[
            pltpu.VMEM((bw,), jnp.int32),
            pltpu.VMEM((bw,), jnp.int32),
            pltpu.VMEM((bw,), jnp.int32),
            pltpu.VMEM((bw,), jnp.int32),
            pltpu.VMEM((bw, 128), jnp.float32),
            pltpu.VMEM((hw, 128), jnp.float32),
            pltpu.SemaphoreType.DMA,
            pltpu.SemaphoreType.DMA,
        ],
    )
    def gather_k(aid_hbm, gid_hbm, atab_hbm, gtab_hbm, aout_hbm, gout_hbm,
                 aidx_v, gidx_v, aslab_v, gslab_v, abuf, gbuf, sem_a, sem_g):
        wid = lax.axis_index("s") * NC + lax.axis_index("c")
        base = wid * bw
        pltpu.sync_copy(aid_hbm.at[pl.ds(base, bw)], aidx_v)
        pltpu.sync_copy(gid_hbm.at[pl.ds(base, bw)], gidx_v)
        for k in range(bw // 16):
            sl = pl.ds(k * 16, 16)
            av = aidx_v[sl]
            aslab_v[sl] = (
                jnp.left_shift(jnp.right_shift(av, 15), 12)
                + jnp.bitwise_and(jnp.right_shift(av, 1), 4095)
            )
            gslab_v[sl] = jnp.right_shift(gidx_v[sl], 2)
        ca = pltpu.async_copy(atab_hbm.at[aslab_v], abuf, sem_a)
        for h in range(2):
            pltpu.async_copy(
                gtab_hbm.at[gslab_v.at[pl.ds(h * hw, hw)]], gbuf, sem_g
            ).wait()
            pltpu.sync_copy(gbuf, gout_hbm.at[pl.ds(base + h * hw, hw)])
        ca.wait()
        pltpu.sync_copy(abuf, aout_hbm.at[pl.ds(base, bw)])

    return gather_k(artist_ids, genre_ids, atab4, gtab4)


def _tc_mlp(a4, g4, meta, y2, wy_row, by_row, w1a, w1g, w1y,
            b1_row, w2, b2_row):
    B = a4.shape[0]
    E = wy_row.shape[1]
    HID = w1a.shape[1]
    OUT = w2.shape[1]
    BLK = 4096

    def mlp_k(a4_ref, g4_ref, meta_ref, y_ref, wy_ref, by_ref,
              w1a_ref, w1g_ref, w1y_ref, b1_ref, w2_ref, b2_ref, o_ref):
        m = jnp.swapaxes(meta_ref[...].reshape(1, BLK), 0, 1)
        asub = jnp.bitwise_and(m, 3)
        parity = jnp.bitwise_and(jnp.right_shift(m, 2), 1)
        gsub = jnp.bitwise_and(jnp.right_shift(m, 3), 3)
        ap = jnp.zeros((BLK, E), jnp.float32)
        g = jnp.zeros((BLK, E), jnp.float32)
        for k in range(4):
            ap = jnp.where(asub == k, a4_ref[:, k * E:(k + 1) * E], ap)
            g = jnp.where(gsub == k, g4_ref[:, k * E:(k + 1) * E], g)
        bits = pltpu.bitcast(ap, jnp.uint32)
        lo = pltpu.bitcast(jnp.left_shift(bits, 16), jnp.float32)
        hi = pltpu.bitcast(
            jnp.bitwise_and(bits, jnp.uint32(0xFFFF0000)), jnp.float32)
        a = jnp.where(parity == 1, hi, lo).astype(jnp.bfloat16)
        gb = g.astype(jnp.bfloat16)
        y_col = jnp.swapaxes(y_ref[...].reshape(1, BLK), 0, 1)
        y_emb = (y_col * wy_ref[...] + by_ref[...]).astype(jnp.bfloat16)
        pre = (
            jnp.dot(a, w1a_ref[...], preferred_element_type=jnp.float32)
            + jnp.dot(gb, w1g_ref[...], preferred_element_type=jnp.float32)
            + jnp.dot(y_emb, w1y_ref[...], preferred_element_type=jnp.float32)
            + b1_ref[...]
        )
        h = jnp.maximum(pre, 0.0).astype(jnp.bfloat16)
        o_ref[...] = jnp.dot(h, w2_ref[...], preferred_element_type=jnp.float32) + b2_ref[...]

    return pl.pallas_call(
        mlp_k,
        grid=(B // BLK,),
        in_specs=[
            pl.BlockSpec((BLK, 128), lambda i: (i, 0)),
            pl.BlockSpec((BLK, 128), lambda i: (i, 0)),
            pl.BlockSpec((1, 1, BLK), lambda i: (i, 0, 0)),
            pl.BlockSpec((1, 1, BLK), lambda i: (i, 0, 0)),
            pl.BlockSpec((1, E), lambda i: (0, 0)),
            pl.BlockSpec((1, E), lambda i: (0, 0)),
            pl.BlockSpec((E, HID), lambda i: (0, 0)),
            pl.BlockSpec((E, HID), lambda i: (0, 0)),
            pl.BlockSpec((E, HID), lambda i: (0, 0)),
            pl.BlockSpec((1, HID), lambda i: (0, 0)),
            pl.BlockSpec((HID, OUT), lambda i: (0, 0)),
            pl.BlockSpec((1, OUT), lambda i: (0, 0)),
        ],
        out_specs=pl.BlockSpec((BLK, OUT), lambda i: (i, 0)),
        out_shape=jax.ShapeDtypeStruct((B, OUT), jnp.float32),
    )(a4, g4, meta, y2, wy_row, by_row, w1a, w1g, w1y, b1_row, w2, b2_row)


def kernel(artist_ids, genre_ids, year_norms, artist_table, genre_table,
           Wy, by, W1, b1, W2, b2):
    E = artist_table.shape[1]
    B = artist_ids.shape[0]
    BLK = 4096
    aid = artist_ids.astype(jnp.int32)
    gid = genre_ids.astype(jnp.int32)
    atab4 = _tc_slabify_bf16(artist_table.T)
    gtab4 = genre_table.reshape(genre_table.shape[0] // 4, 4 * E)
    a4, g4 = _sc_gather_slabs(aid, gid, atab4, gtab4)
    meta = (
        jnp.bitwise_and(jnp.right_shift(aid, 13), 3)
        | jnp.left_shift(jnp.bitwise_and(aid, 1), 2)
        | jnp.left_shift(jnp.bitwise_and(gid, 3), 3)
    ).reshape(B // BLK, 1, BLK)
    y2 = year_norms.reshape(B // BLK, 1, BLK)
    wy_row = Wy.T
    by_row = by[None, :]
    w1a = W1[:, :E].T.astype(jnp.bfloat16)
    w1g = W1[:, E:2 * E].T.astype(jnp.bfloat16)
    w1y = W1[:, 2 * E:3 * E].T.astype(jnp.bfloat16)
    b1_row = b1[None, :]
    w2 = W2.T.astype(jnp.bfloat16)
    b2_row = b2[None, :]
    return _tc_mlp(a4, g4, meta, y2, wy_row, by_row, w1a, w1g, w1y,
                   b1_row, w2, b2_row)
